# baseline (speedup 1.0000x reference)
import functools
import math

import jax
import jax.numpy as jnp
from jax import lax
from jax.experimental import pallas as pl
from jax.experimental.pallas import tpu as pltpu
from jax.experimental.pallas import tpu_sc as plsc

N = 10000
E = 160000
D = 128
NP = 10240
KP = 5120
NCORES = 2
NSUB = 16
CHUNK = 128
SLOT = NCORES * NSUB * CHUNK


@functools.lru_cache(maxsize=None)
def _sc_pass(nchunk, d, nrows, init_from_y, dbuf=True):
    mesh = plsc.VectorSubcoreMesh(core_axis_name="c", subcore_axis_name="s")
    rpt = nrows // NSUB

    def body(y_hbm, rows_hbm, cols_hbm, init_hbm, out_hbm,
             idxr, idxc, buf0, buf1, acc, sem0, sem1):
        c = lax.axis_index("c")
        s = lax.axis_index("s")
        pltpu.sync_copy(
            init_hbm.at[c, pl.ds(s * rpt, rpt)] if init_hbm is not None
            else y_hbm.at[pl.ds(s * rpt, rpt)],
            acc.at[pl.ds(s * rpt, rpt)],
        )
        pltpu.sync_copy(rows_hbm.at[c, s], idxr)
        pltpu.sync_copy(cols_hbm.at[c, s], idxc)
        plsc.subcore_barrier()

        if not dbuf:
            def sstep(j, carry):
                pltpu.async_copy(y_hbm.at[idxr.at[j]], buf0, sem0).wait()
                pltpu.sync_copy(buf0, acc.at[idxc.at[j]], add=True)
                return carry

            lax.fori_loop(0, nchunk, sstep, 0)
        elif nchunk == 1:
            pltpu.async_copy(y_hbm.at[idxr.at[0]], buf0, sem0).wait()
            pltpu.sync_copy(buf0, acc.at[idxc.at[0]], add=True)
        else:
            pltpu.async_copy(y_hbm.at[idxr.at[0]], buf0, sem0)
            def step(jj, carry):
                j0 = 2 * jj
                pltpu.async_copy(y_hbm.at[idxr.at[j0 + 1]], buf1, sem1)
                pltpu.make_async_copy(y_hbm.at[idxr.at[j0]], buf0, sem0).wait()
                pltpu.sync_copy(buf0, acc.at[idxc.at[j0]], add=True)

                @pl.when(j0 + 2 < nchunk)
                def _():
                    pltpu.async_copy(y_hbm.at[idxr.at[j0 + 2]], buf0, sem0)

                pltpu.make_async_copy(
                    y_hbm.at[idxr.at[j0 + 1]], buf1, sem1
                ).wait()
                pltpu.sync_copy(buf1, acc.at[idxc.at[j0 + 1]], add=True)
                return carry

            lax.fori_loop(0, nchunk // 2, step, 0)
        plsc.subcore_barrier()
        pltpu.sync_copy(
            acc.at[pl.ds(s * rpt, rpt)], out_hbm.at[c, pl.ds(s * rpt, rpt)]
        )

    if init_from_y:
        if dbuf:
            def body_wrap(y, r, c_, o, i1, i2, b0, b1, a, s0, s1):
                body(y, r, c_, None, o, i1, i2, b0, b1, a, s0, s1)
        else:
            def body_wrap(y, r, c_, o, i1, i2, b0, a, s0):
                body(y, r, c_, None, o, i1, i2, b0, None, a, s0, None)
    else:
        if dbuf:
            body_wrap = body
        else:
            def body_wrap(y, r, c_, init, o, i1, i2, b0, a, s0):
                body(y, r, c_, init, o, i1, i2, b0, None, a, s0, None)

    scratch_types = [
        pltpu.VMEM((nchunk, CHUNK), jnp.int32),
        pltpu.VMEM((nchunk, CHUNK), jnp.int32),
        pltpu.VMEM((CHUNK, d), jnp.float32),
    ]
    if dbuf:
        scratch_types.append(pltpu.VMEM((CHUNK, d), jnp.float32))
    scratch_types.append(pltpu.VMEM_SHARED((nrows, d), jnp.float32))
    scratch_types.append(pltpu.SemaphoreType.DMA)
    if dbuf:
        scratch_types.append(pltpu.SemaphoreType.DMA)

    return pl.kernel(
        body_wrap,
        mesh=mesh,
        out_type=jax.ShapeDtypeStruct((NCORES, nrows, d), jnp.float32),
        scratch_types=scratch_types,
    )


def _pack_edges(r, c, nchunk, dump=N):
    total = SLOT * nchunk
    pad = total - r.shape[0]
    rp = jnp.concatenate([r, jnp.full((pad,), dump, jnp.int32)])
    cp = jnp.concatenate([c, jnp.full((pad,), dump, jnp.int32)])
    return (
        rp.reshape(NCORES, NSUB, nchunk, CHUNK),
        cp.reshape(NCORES, NSUB, nchunk, CHUNK),
    )


def _mm_body(n_in, *refs):
    rsin_ref, w_ref, rsout_ref, o_ref = refs[n_in:]
    xs = refs[0][...]
    for r in refs[1:n_in]:
        xs = xs + r[...]
    xb = xs * rsin_ref[...][:, None]
    acc = jnp.dot(
        xb,
        w_ref[...],
        preferred_element_type=jnp.float32,
        precision=lax.Precision.HIGHEST,
    )
    o_ref[...] = acc * rsout_ref[...][:, None]


@functools.lru_cache(maxsize=None)
def _mm(m, n_in=2, blk=1024):
    grid = m // blk
    mat = pl.BlockSpec((blk, D), lambda i: (i, 0))
    vec = pl.BlockSpec((blk,), lambda i: (i,))
    in_specs = [mat] * n_in + [
        vec,
        pl.BlockSpec((D, D), lambda i: (0, 0)),
        vec,
    ]
    return pl.pallas_call(
        functools.partial(_mm_body, n_in),
        grid=(grid,),
        in_specs=in_specs,
        out_specs=mat,
        out_shape=jax.ShapeDtypeStruct((m, D), jnp.float32),
    )


def _ep_body(do_tanh, s0_ref, s1_ref, hp_ref, dinv_ref, a_ref, b_ref, pn_ref,
             x_ref, sc_ref):
    i = pl.program_id(0)
    blk = x_ref.shape[0]
    v = (
        dinv_ref[...][:, None] * (s0_ref[...] + s1_ref[...])
        + a_ref[...][:, None] * hp_ref[...]
        + b_ref[...][None, :]
    )
    if do_tanh:
        v = jnp.tanh(v)
    rows = i * blk + lax.broadcasted_iota(jnp.int32, (blk, 1), 0)
    v = jnp.where(rows < N, v, 0.0)
    x_ref[...] = v
    sc = jnp.tanh(
        jnp.dot(
            v,
            pn_ref[...][:, None],
            preferred_element_type=jnp.float32,
            precision=lax.Precision.HIGHEST,
        )
    )[:, 0]
    sc_ref[...] = jnp.where(rows[:, 0] < N, sc, -2.0)


@functools.lru_cache(maxsize=None)
def _ep(do_tanh, blk=1024):
    grid = NP // blk
    return pl.pallas_call(
        functools.partial(_ep_body, do_tanh),
        grid=(grid,),
        in_specs=[
            pl.BlockSpec((blk, D), lambda i: (i, 0)),
            pl.BlockSpec((blk, D), lambda i: (i, 0)),
            pl.BlockSpec((blk, D), lambda i: (i, 0)),
            pl.BlockSpec((blk,), lambda i: (i,)),
            pl.BlockSpec((blk,), lambda i: (i,)),
            pl.BlockSpec((D,), lambda i: (0,)),
            pl.BlockSpec((D,), lambda i: (0,)),
        ],
        out_specs=[
            pl.BlockSpec((blk, D), lambda i: (i, 0)),
            pl.BlockSpec((blk,), lambda i: (i,)),
        ],
        out_shape=[
            jax.ShapeDtypeStruct((NP, D), jnp.float32),
            jax.ShapeDtypeStruct((NP,), jnp.float32),
        ],
    )


def _ep2_body(y0_ref, y1_ref, g_ref, dinv_ref, db_ref, b_ref, o_ref):
    g = g_ref[...]
    v = y0_ref[...] + y1_ref[...] + (2.0 - db_ref[...][:, None]) * g
    o_ref[...] = jnp.tanh(dinv_ref[...][:, None] * v + b_ref[...][None, :])


@functools.lru_cache(maxsize=None)
def _ep2(blk=1024):
    grid = KP // blk
    return pl.pallas_call(
        _ep2_body,
        grid=(grid,),
        in_specs=[
            pl.BlockSpec((blk, D), lambda i: (i, 0)),
            pl.BlockSpec((blk, D), lambda i: (i, 0)),
            pl.BlockSpec((blk, D), lambda i: (i, 0)),
            pl.BlockSpec((blk,), lambda i: (i,)),
            pl.BlockSpec((blk,), lambda i: (i,)),
            pl.BlockSpec((D,), lambda i: (0,)),
        ],
        out_specs=pl.BlockSpec((blk, D), lambda i: (i, 0)),
        out_shape=jax.ShapeDtypeStruct((KP, D), jnp.float32),
    )


def _nc(n_edges):
    nc = -(-n_edges // SLOT)
    return nc if nc <= 1 else nc + (nc & 1)


def kernel(x, edge_index, W0, b0, p0, W1, b1, Wu, bu):
    f32 = jnp.float32
    row = edge_index[0].astype(jnp.int32)
    col = edge_index[1].astype(jnp.int32)
    off = row != col
    zeros2 = jnp.zeros((NCORES, NP, D), f32)
    ones_np = jnp.ones(NP, f32)

    nc_full = _nc(E)
    nc_perm = _nc(KP)
    er_full, ec_full = _pack_edges(row, col, nc_full)
    rm = jnp.where(off, row, N)
    cm = jnp.where(off, col, N)
    er_mask, ec_mask = _pack_edges(rm, cm, nc_full)

    onehot = jnp.zeros((8, D), f32).at[0, 0].set(1.0).at[1, 1].set(1.0)
    nc_ds = _nc(2 * E)
    ds_src = jnp.concatenate(
        [jnp.zeros(E, jnp.int32), jnp.where(off, 2, 1).astype(jnp.int32)]
    )
    ds_dst = jnp.concatenate([col, jnp.where(off, N, row)])
    ds_r, ds_c = _pack_edges(ds_src, ds_dst, nc_ds)
    ds_r = jnp.where(ds_r == N, 2, ds_r)
    hist = _sc_pass(nc_ds, D, NP, False, dbuf=False)(onehot, ds_r, ds_c, zeros2)
    hist = hist[0] + hist[1]
    deg0 = hist[:N, 0]
    self_cnt = hist[:N, 1]

    addw = jnp.where(self_cnt > 0, 0.0, 2.0).astype(f32)
    deg0 = deg0 + addw
    dinv0 = jnp.where(deg0 > 0, deg0 ** -0.5, 0.0).astype(f32)
    dinv0_p = jnp.pad(dinv0, (0, NP - N))
    addw_p = jnp.pad(addw, (0, NP - N))

    xpad = jnp.pad(x, ((0, NP - N), (0, 0)))
    h0p = _mm(NP, n_in=1)(xpad, ones_np, W0, dinv0_p)
    s_parts = _sc_pass(nc_full, D, NP, False)(h0p, er_full, ec_full, zeros2)
    p0n = (p0 / jnp.linalg.norm(p0)).astype(f32)
    x1, score = _ep(True)(
        s_parts[0], s_parts[1], h0p, dinv0_p, addw_p * dinv0_p, b0, p0n
    )

    k = int(math.ceil(0.5 * N))
    _, perm = lax.top_k(score, k)
    perm = perm.astype(jnp.int32)
    perm_p = jnp.concatenate([perm, jnp.full((KP - k,), N, jnp.int32)])

    key = jnp.where(off, row * N + col, N * N + row)
    qry = col * N + row
    v = jnp.sort(jnp.concatenate([2 * key, 2 * qry + 1]))
    is_key = (v & 1) == 0
    kcum = jnp.cumsum(is_key.astype(jnp.int32))
    excl = kcum - is_key.astype(jnp.int32)
    grp = v >> 1
    newg = jnp.concatenate(
        [jnp.ones((1,), jnp.bool_), grp[1:] != grp[:-1]]
    )
    base = lax.cummax(jnp.where(newg, excl, -1), axis=0)
    cnt = (kcum - base).astype(f32)
    q = (v - 1) >> 1
    qc = q // N
    qr = q - qc * N
    is_q = jnp.logical_and(jnp.logical_not(is_key), qr != qc)
    rc_dst = jnp.where(is_q, qr, N)
    dBe = jnp.zeros(N + 1, f32).at[rc_dst].add(cnt)[:N]

    m_ind = jnp.pad(jnp.zeros(N, f32).at[perm].set(1.0), (0, NP - N))
    m_mat = jnp.broadcast_to(m_ind[:, None], (NP, D)) + jnp.zeros((NP, D), f32)
    c1p = _sc_pass(nc_full, D, NP, True)(m_mat, er_mask, ec_mask)
    c1m = c1p[0] + c1p[1] - m_mat
    c2p = _sc_pass(nc_full, D, NP, True)(c1m, er_mask, ec_mask)
    c2 = (c2p[0] + c2p[1] - c1m)[:N, 0]
    dBp = dBe[perm] + 1.0
    deg1 = c2[perm] - dBp + 2.0
    dinv1 = jnp.where(deg1 > 0, deg1 ** -0.5, 0.0).astype(f32)
    dinv1_p = jnp.pad(dinv1, (0, KP - k))
    dBp_p = jnp.pad(dBp, (0, KP - k))

    iota_kp = jnp.arange(KP, dtype=jnp.int32)
    gr_r, gr_c = _pack_edges(perm_p, iota_kp, nc_perm)
    sc_r, sc_c = _pack_edges(iota_kp, perm_p, nc_perm)
    gparts = _sc_pass(nc_perm, D, NP, False)(x1, gr_r, gr_c, zeros2)
    sperm_p = jnp.pad(score[perm], (0, KP - k))
    g = _mm(KP)(gparts[0, :KP], gparts[1, :KP], sperm_p, W1, dinv1_p)
    g_np = jnp.pad(g, ((0, NP - KP), (0, 0)))
    gf_parts = _sc_pass(nc_perm, D, NP, False)(g_np, sc_r, sc_c, zeros2)
    Gf = gf_parts[0] + gf_parts[1]
    y1_parts = _sc_pass(nc_full, D, NP, True)(Gf, er_mask, ec_mask)
    Y1 = y1_parts[0] + y1_parts[1] - Gf
    y2_parts = _sc_pass(nc_full, D, NP, True)(Y1, er_mask, ec_mask)
    Y2 = y2_parts[0] + y2_parts[1] - Y1
    y2p_parts = _sc_pass(nc_perm, D, NP, False)(Y2, gr_r, gr_c, zeros2)
    x2 = _ep2()(y2p_parts[0, :KP], y2p_parts[1, :KP], g, dinv1_p, dBp_p, b1)

    x2_np = jnp.pad(x2, ((0, NP - KP), (0, 0)))
    up_parts = _sc_pass(nc_perm, D, NP, False)(x2_np, sc_r, sc_c, zeros2)
    hup = _mm(NP, n_in=3)(up_parts[0], up_parts[1], x1, ones_np, Wu, dinv0_p)
    su_parts = _sc_pass(nc_full, D, NP, False)(hup, er_full, ec_full, zeros2)
    out_full, _ = _ep(False)(
        su_parts[0], su_parts[1], hup, dinv0_p, addw_p * dinv0_p, bu, p0n
    )
    return out_full[:N]


# speedup vs baseline: 2.9088x; 2.9088x over previous
import functools
import math

import jax
import jax.numpy as jnp
from jax import lax
from jax.experimental import pallas as pl
from jax.experimental.pallas import tpu as pltpu
from jax.experimental.pallas import tpu_sc as plsc

N = 10000
E = 160000
D = 128
NP = 10240
KP = 5120
NCORES = 2
NSUB = 16
CHUNK = 128
SLOT = NCORES * NSUB * CHUNK


@functools.lru_cache(maxsize=None)
def _sc_pass(nchunk, d, nrows, init_from_y, dbuf=True):
    mesh = plsc.VectorSubcoreMesh(core_axis_name="c", subcore_axis_name="s")
    rpt = nrows // NSUB

    def body(y_hbm, rows_hbm, cols_hbm, init_hbm, out_hbm,
             idxr, idxc, buf0, buf1, acc, sem0, sem1):
        c = lax.axis_index("c")
        s = lax.axis_index("s")
        pltpu.sync_copy(
            init_hbm.at[c, pl.ds(s * rpt, rpt)] if init_hbm is not None
            else y_hbm.at[pl.ds(s * rpt, rpt)],
            acc.at[pl.ds(s * rpt, rpt)],
        )
        pltpu.sync_copy(rows_hbm.at[c, s], idxr)
        pltpu.sync_copy(cols_hbm.at[c, s], idxc)
        plsc.subcore_barrier()

        if not dbuf:
            def sstep(j, carry):
                pltpu.async_copy(y_hbm.at[idxr.at[j]], buf0, sem0).wait()
                pltpu.sync_copy(buf0, acc.at[idxc.at[j]], add=True)
                return carry

            lax.fori_loop(0, nchunk, sstep, 0)
        elif nchunk == 1:
            pltpu.async_copy(y_hbm.at[idxr.at[0]], buf0, sem0).wait()
            pltpu.sync_copy(buf0, acc.at[idxc.at[0]], add=True)
        else:
            pltpu.async_copy(y_hbm.at[idxr.at[0]], buf0, sem0)
            def step(jj, carry):
                j0 = 2 * jj
                pltpu.async_copy(y_hbm.at[idxr.at[j0 + 1]], buf1, sem1)
                pltpu.make_async_copy(y_hbm.at[idxr.at[j0]], buf0, sem0).wait()
                pltpu.sync_copy(buf0, acc.at[idxc.at[j0]], add=True)

                @pl.when(j0 + 2 < nchunk)
                def _():
                    pltpu.async_copy(y_hbm.at[idxr.at[j0 + 2]], buf0, sem0)

                pltpu.make_async_copy(
                    y_hbm.at[idxr.at[j0 + 1]], buf1, sem1
                ).wait()
                pltpu.sync_copy(buf1, acc.at[idxc.at[j0 + 1]], add=True)
                return carry

            lax.fori_loop(0, nchunk // 2, step, 0)
        plsc.subcore_barrier()
        pltpu.sync_copy(
            acc.at[pl.ds(s * rpt, rpt)], out_hbm.at[c, pl.ds(s * rpt, rpt)]
        )

    if init_from_y:
        if dbuf:
            def body_wrap(y, r, c_, o, i1, i2, b0, b1, a, s0, s1):
                body(y, r, c_, None, o, i1, i2, b0, b1, a, s0, s1)
        else:
            def body_wrap(y, r, c_, o, i1, i2, b0, a, s0):
                body(y, r, c_, None, o, i1, i2, b0, None, a, s0, None)
    else:
        if dbuf:
            body_wrap = body
        else:
            def body_wrap(y, r, c_, init, o, i1, i2, b0, a, s0):
                body(y, r, c_, init, o, i1, i2, b0, None, a, s0, None)

    scratch_types = [
        pltpu.VMEM((nchunk, CHUNK), jnp.int32),
        pltpu.VMEM((nchunk, CHUNK), jnp.int32),
        pltpu.VMEM((CHUNK, d), jnp.float32),
    ]
    if dbuf:
        scratch_types.append(pltpu.VMEM((CHUNK, d), jnp.float32))
    scratch_types.append(pltpu.VMEM_SHARED((nrows, d), jnp.float32))
    scratch_types.append(pltpu.SemaphoreType.DMA)
    if dbuf:
        scratch_types.append(pltpu.SemaphoreType.DMA)

    return pl.kernel(
        body_wrap,
        mesh=mesh,
        out_type=jax.ShapeDtypeStruct((NCORES, nrows, d), jnp.float32),
        scratch_types=scratch_types,
    )


def _pack_edges(r, c, nchunk, dump=N):
    total = SLOT * nchunk
    pad = total - r.shape[0]
    rp = jnp.concatenate([r, jnp.full((pad,), dump, jnp.int32)])
    cp = jnp.concatenate([c, jnp.full((pad,), dump, jnp.int32)])
    return (
        rp.reshape(NCORES, NSUB, nchunk, CHUNK),
        cp.reshape(NCORES, NSUB, nchunk, CHUNK),
    )


def _mm_body(n_in, *refs):
    rsin_ref, w_ref, rsout_ref, o_ref = refs[n_in:]
    xs = refs[0][...]
    for r in refs[1:n_in]:
        xs = xs + r[...]
    xb = xs * rsin_ref[...][:, None]
    acc = jnp.dot(
        xb,
        w_ref[...],
        preferred_element_type=jnp.float32,
        precision=lax.Precision.HIGHEST,
    )
    o_ref[...] = acc * rsout_ref[...][:, None]


@functools.lru_cache(maxsize=None)
def _mm(m, n_in=2, blk=1024):
    grid = m // blk
    mat = pl.BlockSpec((blk, D), lambda i: (i, 0))
    vec = pl.BlockSpec((blk,), lambda i: (i,))
    in_specs = [mat] * n_in + [
        vec,
        pl.BlockSpec((D, D), lambda i: (0, 0)),
        vec,
    ]
    return pl.pallas_call(
        functools.partial(_mm_body, n_in),
        grid=(grid,),
        in_specs=in_specs,
        out_specs=mat,
        out_shape=jax.ShapeDtypeStruct((m, D), jnp.float32),
    )


def _ep_body(do_tanh, s0_ref, s1_ref, hp_ref, dinv_ref, a_ref, b_ref, pn_ref,
             x_ref, sc_ref):
    i = pl.program_id(0)
    blk = x_ref.shape[0]
    v = (
        dinv_ref[...][:, None] * (s0_ref[...] + s1_ref[...])
        + a_ref[...][:, None] * hp_ref[...]
        + b_ref[...][None, :]
    )
    if do_tanh:
        v = jnp.tanh(v)
    rows = i * blk + lax.broadcasted_iota(jnp.int32, (blk, 1), 0)
    v = jnp.where(rows < N, v, 0.0)
    x_ref[...] = v
    sc = jnp.tanh(
        jnp.dot(
            v,
            pn_ref[...][:, None],
            preferred_element_type=jnp.float32,
            precision=lax.Precision.HIGHEST,
        )
    )[:, 0]
    sc_ref[...] = jnp.where(rows[:, 0] < N, sc, -2.0)


@functools.lru_cache(maxsize=None)
def _ep(do_tanh, blk=1024):
    grid = NP // blk
    return pl.pallas_call(
        functools.partial(_ep_body, do_tanh),
        grid=(grid,),
        in_specs=[
            pl.BlockSpec((blk, D), lambda i: (i, 0)),
            pl.BlockSpec((blk, D), lambda i: (i, 0)),
            pl.BlockSpec((blk, D), lambda i: (i, 0)),
            pl.BlockSpec((blk,), lambda i: (i,)),
            pl.BlockSpec((blk,), lambda i: (i,)),
            pl.BlockSpec((D,), lambda i: (0,)),
            pl.BlockSpec((D,), lambda i: (0,)),
        ],
        out_specs=[
            pl.BlockSpec((blk, D), lambda i: (i, 0)),
            pl.BlockSpec((blk,), lambda i: (i,)),
        ],
        out_shape=[
            jax.ShapeDtypeStruct((NP, D), jnp.float32),
            jax.ShapeDtypeStruct((NP,), jnp.float32),
        ],
    )


def _ep2_body(y0_ref, y1_ref, g_ref, dinv_ref, db_ref, b_ref, o_ref):
    g = g_ref[...]
    v = y0_ref[...] + y1_ref[...] + (2.0 - db_ref[...][:, None]) * g
    o_ref[...] = jnp.tanh(dinv_ref[...][:, None] * v + b_ref[...][None, :])


@functools.lru_cache(maxsize=None)
def _ep2(blk=1024):
    grid = KP // blk
    return pl.pallas_call(
        _ep2_body,
        grid=(grid,),
        in_specs=[
            pl.BlockSpec((blk, D), lambda i: (i, 0)),
            pl.BlockSpec((blk, D), lambda i: (i, 0)),
            pl.BlockSpec((blk, D), lambda i: (i, 0)),
            pl.BlockSpec((blk,), lambda i: (i,)),
            pl.BlockSpec((blk,), lambda i: (i,)),
            pl.BlockSpec((D,), lambda i: (0,)),
        ],
        out_specs=pl.BlockSpec((blk, D), lambda i: (i, 0)),
        out_shape=jax.ShapeDtypeStruct((KP, D), jnp.float32),
    )


def _nc(n_edges):
    nc = -(-n_edges // SLOT)
    return nc if nc <= 1 else nc + (nc & 1)


def kernel(x, edge_index, W0, b0, p0, W1, b1, Wu, bu):
    f32 = jnp.float32
    row = edge_index[0].astype(jnp.int32)
    col = edge_index[1].astype(jnp.int32)
    off = row != col
    zeros2 = jnp.zeros((NCORES, NP, D), f32)
    ones_np = jnp.ones(NP, f32)

    nc_full = _nc(E)
    nc_perm = _nc(KP)
    er_full, ec_full = _pack_edges(row, col, nc_full)
    rm = jnp.where(off, row, N)
    cm = jnp.where(off, col, N)
    er_mask, ec_mask = _pack_edges(rm, cm, nc_full)

    ds_idx = jnp.concatenate([col, jnp.where(off, jnp.int32(2 * N), N + row)])
    degself = jnp.zeros(2 * N + 1, f32).at[ds_idx].add(1.0)
    deg0 = degself[:N]
    self_cnt = degself[N:2 * N]

    addw = jnp.where(self_cnt > 0, 0.0, 2.0).astype(f32)
    deg0 = deg0 + addw
    dinv0 = jnp.where(deg0 > 0, deg0 ** -0.5, 0.0).astype(f32)
    dinv0_p = jnp.pad(dinv0, (0, NP - N))
    addw_p = jnp.pad(addw, (0, NP - N))

    xpad = jnp.pad(x, ((0, NP - N), (0, 0)))
    h0p = _mm(NP, n_in=1)(xpad, ones_np, W0, dinv0_p)
    s_parts = _sc_pass(nc_full, D, NP, False)(h0p, er_full, ec_full, zeros2)
    p0n = (p0 / jnp.linalg.norm(p0)).astype(f32)
    x1, score = _ep(True)(
        s_parts[0], s_parts[1], h0p, dinv0_p, addw_p * dinv0_p, b0, p0n
    )

    k = int(math.ceil(0.5 * N))
    _, perm = lax.top_k(score, k)
    perm = perm.astype(jnp.int32)
    perm_p = jnp.concatenate([perm, jnp.full((KP - k,), N, jnp.int32)])

    key = jnp.where(off, row * N + col, N * N + row)
    qry = col * N + row
    v = jnp.sort(jnp.concatenate([2 * key, 2 * qry + 1]))
    is_key = (v & 1) == 0
    kcum = jnp.cumsum(is_key.astype(jnp.int32))
    excl = kcum - is_key.astype(jnp.int32)
    grp = v >> 1
    newg = jnp.concatenate(
        [jnp.ones((1,), jnp.bool_), grp[1:] != grp[:-1]]
    )
    base = lax.cummax(jnp.where(newg, excl, -1), axis=0)
    cnt = (kcum - base).astype(f32)
    q = (v - 1) >> 1
    qc = q // N
    qr = q - qc * N
    is_q = jnp.logical_and(jnp.logical_not(is_key), qr != qc)
    rc_dst = jnp.where(is_q, qr, N)
    dBe = jnp.zeros(N + 1, f32).at[rc_dst].add(cnt)[:N]

    m_ind = jnp.pad(jnp.zeros(N, f32).at[perm].set(1.0), (0, NP - N))
    m_mat = jnp.broadcast_to(m_ind[:, None], (NP, D)) + jnp.zeros((NP, D), f32)
    c1p = _sc_pass(nc_full, D, NP, True)(m_mat, er_mask, ec_mask)
    c1m = c1p[0] + c1p[1] - m_mat
    c2p = _sc_pass(nc_full, D, NP, True)(c1m, er_mask, ec_mask)
    c2 = (c2p[0] + c2p[1] - c1m)[:N, 0]
    dBp = dBe[perm] + 1.0
    deg1 = c2[perm] - dBp + 2.0
    dinv1 = jnp.where(deg1 > 0, deg1 ** -0.5, 0.0).astype(f32)
    dinv1_p = jnp.pad(dinv1, (0, KP - k))
    dBp_p = jnp.pad(dBp, (0, KP - k))

    iota_kp = jnp.arange(KP, dtype=jnp.int32)
    gr_r, gr_c = _pack_edges(perm_p, iota_kp, nc_perm)
    sc_r, sc_c = _pack_edges(iota_kp, perm_p, nc_perm)
    gparts = _sc_pass(nc_perm, D, NP, False)(x1, gr_r, gr_c, zeros2)
    sperm_p = jnp.pad(score[perm], (0, KP - k))
    g = _mm(KP)(gparts[0, :KP], gparts[1, :KP], sperm_p, W1, dinv1_p)
    g_np = jnp.pad(g, ((0, NP - KP), (0, 0)))
    gf_parts = _sc_pass(nc_perm, D, NP, False)(g_np, sc_r, sc_c, zeros2)
    Gf = gf_parts[0] + gf_parts[1]
    y1_parts = _sc_pass(nc_full, D, NP, True)(Gf, er_mask, ec_mask)
    Y1 = y1_parts[0] + y1_parts[1] - Gf
    y2_parts = _sc_pass(nc_full, D, NP, True)(Y1, er_mask, ec_mask)
    Y2 = y2_parts[0] + y2_parts[1] - Y1
    y2p_parts = _sc_pass(nc_perm, D, NP, False)(Y2, gr_r, gr_c, zeros2)
    x2 = _ep2()(y2p_parts[0, :KP], y2p_parts[1, :KP], g, dinv1_p, dBp_p, b1)

    x2_np = jnp.pad(x2, ((0, NP - KP), (0, 0)))
    up_parts = _sc_pass(nc_perm, D, NP, False)(x2_np, sc_r, sc_c, zeros2)
    hup = _mm(NP, n_in=3)(up_parts[0], up_parts[1], x1, ones_np, Wu, dinv0_p)
    su_parts = _sc_pass(nc_full, D, NP, False)(hup, er_full, ec_full, zeros2)
    out_full, _ = _ep(False)(
        su_parts[0], su_parts[1], hup, dinv0_p, addw_p * dinv0_p, bu, p0n
    )
    return out_full[:N]
